# baseline (speedup 1.0000x reference)
import functools

import jax
import jax.numpy as jnp
from jax import lax
from jax.experimental import pallas as pl
from jax.experimental.pallas import tpu as pltpu
from jax.experimental.pallas import tpu_sc as plsc

L = 16
N = 4
C = 384
HW = 224 * 224
ROWS = N * C
NC = 2
NS = 16
NW = NC * NS
RPW = ROWS // NW


def _sc_body(inp_hbm, mask_hbm, out_hbm,
             mask_v, sel_v, idx_v, buf0, buf1,
             sem_in0, sem_in1, sem_out0, sem_out1):
    cid = lax.axis_index("c")
    sid = lax.axis_index("s")
    wid = sid * NC + cid

    pltpu.sync_copy(mask_hbm, mask_v)
    zeros = jnp.zeros((L,), jnp.int32)
    for k in range(C // L):
        sel_v[pl.ds(k * L, L)] = zeros
    count = jnp.int32(0)
    for k in range(C // L):
        v = mask_v[pl.ds(k * L, L)]
        nz = v != 0.0
        nzi = nz.astype(jnp.int32)
        cs = plsc.cumsum(nzi)
        pos = count + cs - nzi
        vals = lax.iota(jnp.int32, L) + (k * L)
        plsc.store_scatter(sel_v, [pos], vals, mask=nz)
        count = count + jnp.sum(nzi)

    n_img = wid // (C // RPW)
    j0 = (wid % (C // RPW)) * RPW
    base = n_img * C
    for k in range(RPW // L):
        jvec = lax.iota(jnp.int32, L) + (j0 + k * L)
        rows = plsc.load_gather(sel_v, [jvec]) + base
        pos = (lax.iota(jnp.int32, L) + k * L) * 8
        plsc.store_scatter(idx_v, [pos], rows)

    out0 = wid * RPW
    bufs = (buf0, buf1)
    in_sems = (sem_in0, sem_in1)
    out_sems = (sem_out0, sem_out1)

    def start_gather(t):
        return pltpu.async_copy(
            inp_hbm.at[idx_v.at[pl.ds(8 * t, 1)]], bufs[t % 2], in_sems[t % 2])

    def start_write(t):
        return pltpu.async_copy(
            bufs[t % 2], out_hbm.at[pl.ds(out0 + t, 1)], out_sems[t % 2])

    gathers = [None] * RPW
    writes = [None] * RPW
    gathers[0] = start_gather(0)
    for t in range(RPW):
        if t + 1 < RPW:
            if t - 1 >= 0:
                writes[t - 1].wait()
            gathers[t + 1] = start_gather(t + 1)
        gathers[t].wait()
        writes[t] = start_write(t)
    writes[RPW - 2].wait()
    writes[RPW - 1].wait()


@jax.jit
def _sc_gather(inp2, mask):
    mesh = plsc.VectorSubcoreMesh(core_axis_name="c", subcore_axis_name="s",
                                  num_cores=NC, num_subcores=NS)
    return pl.kernel(
        _sc_body,
        out_type=jax.ShapeDtypeStruct((ROWS, HW), jnp.float32),
        mesh=mesh,
        compiler_params=pltpu.CompilerParams(needs_layout_passes=False),
        scratch_types=[
            pltpu.VMEM((C,), jnp.float32),
            pltpu.VMEM((C,), jnp.int32),
            pltpu.VMEM((RPW * 8,), jnp.int32),
            pltpu.VMEM((1, HW), jnp.float32),
            pltpu.VMEM((1, HW), jnp.float32),
            pltpu.SemaphoreType.DMA,
            pltpu.SemaphoreType.DMA,
            pltpu.SemaphoreType.DMA,
            pltpu.SemaphoreType.DMA,
        ],
    )(inp2, mask)


def kernel(input_tensor, indexes):
    inp2 = input_tensor.reshape(ROWS, HW)
    out2 = _sc_gather(inp2, indexes)
    return out2.reshape(input_tensor.shape)


# speedup vs baseline: 1.5485x; 1.5485x over previous
import jax
import jax.numpy as jnp
from jax import lax
from jax.experimental import pallas as pl
from jax.experimental.pallas import tpu as pltpu
from jax.experimental.pallas import tpu_sc as plsc

L = 16
N = 4
C = 384
H = 224
W = 224
NC = 2
NS = 16
NW = NC * NS
PPW = N * C // NW
WPI = C // PPW


def _sc_body(inp_hbm, mask_hbm, out_hbm,
             mask_v, sel_v, buf0, buf1,
             sem_in0, sem_in1, sem_out0, sem_out1):
    cid = lax.axis_index("c")
    sid = lax.axis_index("s")
    wid = sid * NC + cid

    pltpu.sync_copy(mask_hbm, mask_v)
    zeros = jnp.zeros((L,), jnp.int32)
    for k in range(C // L):
        sel_v[pl.ds(k * L, L)] = zeros
    count = jnp.int32(0)
    for k in range(C // L):
        v = mask_v[pl.ds(k * L, L)]
        nz = v != 0.0
        nzi = nz.astype(jnp.int32)
        cs = plsc.cumsum(nzi)
        pos = count + cs - nzi
        vals = lax.iota(jnp.int32, L) + (k * L)
        plsc.store_scatter(sel_v, [pos], vals, mask=nz)
        count = count + jnp.sum(nzi)

    n_img = wid // WPI
    j0 = (wid % WPI) * PPW
    bufs = (buf0, buf1)
    in_sems = (sem_in0, sem_in1)
    out_sems = (sem_out0, sem_out1)

    def start_read(t):
        grp = sel_v[pl.ds(pl.multiple_of(j0 + (t // L) * L, 8), L)]
        src_c = grp[t % L]
        return pltpu.async_copy(
            inp_hbm.at[n_img, pl.ds(src_c, 1)], bufs[t % 2], in_sems[t % 2])

    def start_write(t):
        return pltpu.async_copy(
            bufs[t % 2], out_hbm.at[n_img, pl.ds(j0 + t, 1)], out_sems[t % 2])

    reads = [None] * PPW
    writes = [None] * PPW
    reads[0] = start_read(0)
    for t in range(PPW):
        if t + 1 < PPW:
            if t - 1 >= 0:
                writes[t - 1].wait()
            reads[t + 1] = start_read(t + 1)
        reads[t].wait()
        writes[t] = start_write(t)
    writes[PPW - 2].wait()
    writes[PPW - 1].wait()


@jax.jit
def _sc_gather(inp, mask):
    mesh = plsc.VectorSubcoreMesh(core_axis_name="c", subcore_axis_name="s",
                                  num_cores=NC, num_subcores=NS)
    return pl.kernel(
        _sc_body,
        out_type=jax.ShapeDtypeStruct((N, C, H, W), jnp.float32),
        mesh=mesh,
        compiler_params=pltpu.CompilerParams(needs_layout_passes=False),
        scratch_types=[
            pltpu.VMEM((C,), jnp.float32),
            pltpu.VMEM((C,), jnp.int32),
            pltpu.VMEM((1, H, W), jnp.float32),
            pltpu.VMEM((1, H, W), jnp.float32),
            pltpu.SemaphoreType.DMA,
            pltpu.SemaphoreType.DMA,
            pltpu.SemaphoreType.DMA,
            pltpu.SemaphoreType.DMA,
        ],
    )(inp, mask)


def kernel(input_tensor, indexes):
    return _sc_gather(input_tensor, indexes)


# speedup vs baseline: 5.8311x; 3.7656x over previous
import jax
import jax.numpy as jnp
from jax import lax
from jax.experimental import pallas as pl
from jax.experimental.pallas import tpu as pltpu
from jax.experimental.pallas import tpu_sc as plsc

L = 16
N = 4
C = 384
H = 224
W = 224
PIX = N * H * W
NC = 2
NS = 16
NW = NC * NS
PPW = PIX // NW
NBUF = 4
GCHUNK = 64


def _sc_body(inp_hbm, mask_hbm, out_hbm, mask_v, sel_v,
             buf0, buf1, buf2, buf3,
             isem0, isem1, isem2, isem3, osem0, osem1, osem2, osem3):
    cid = lax.axis_index("c")
    sid = lax.axis_index("s")
    wid = sid * NC + cid

    pltpu.sync_copy(mask_hbm, mask_v)
    zeros = jnp.zeros((L,), jnp.int32)
    for k in range(C // L):
        sel_v[pl.ds(k * L, L)] = zeros
    count = jnp.int32(0)
    mism = jnp.int32(0)
    for k in range(C // L):
        v = mask_v[pl.ds(k * L, L)]
        nz = v != 0.0
        nzi = nz.astype(jnp.int32)
        cs = plsc.cumsum(nzi)
        pos = count + cs - nzi
        vals = lax.iota(jnp.int32, L) + (k * L)
        plsc.store_scatter(sel_v, [pos], vals, mask=nz)
        count = count + jnp.sum(nzi)
        mism = mism + jnp.sum((~nz).astype(jnp.int32))

    base = wid * PPW
    bufs = (buf0, buf1, buf2, buf3)
    in_sems = (isem0, isem1, isem2, isem3)
    out_sems = (osem0, osem1, osem2, osem3)
    nq = PPW // GCHUNK

    def start_read(q):
        return pltpu.async_copy(
            inp_hbm.at[pl.ds(base + q * GCHUNK, GCHUNK)],
            bufs[q % NBUF], in_sems[q % NBUF])

    def start_write(q):
        return pltpu.async_copy(
            bufs[q % NBUF],
            out_hbm.at[pl.ds(base + q * GCHUNK, GCHUNK)],
            out_sems[q % NBUF])

    @pl.when(mism == 0)
    def _fast():
        reads = [None] * nq
        writes = [None] * nq
        for q in range(min(NBUF - 1, nq)):
            reads[q] = start_read(q)
        for q in range(nq):
            if q + NBUF - 1 < nq:
                if q - 1 >= 0:
                    writes[q - 1].wait()
                reads[q + NBUF - 1] = start_read(q + NBUF - 1)
            reads[q].wait()
            writes[q] = start_write(q)
        for r in range(max(0, nq - NBUF), nq):
            writes[r].wait()

    @pl.when(mism != 0)
    def _general():
        def chunk_body(q, carry):
            lo = base + q * GCHUNK
            pltpu.sync_copy(inp_hbm.at[pl.ds(lo, GCHUNK)], buf0)

            def pixel_body(p, c2):
                for g in range(C // L):
                    off = pl.multiple_of(g * L, 8)
                    src_c = sel_v[pl.ds(off, L)]
                    rows = jnp.zeros((L,), jnp.int32) + p
                    vals = plsc.load_gather(buf0, [rows, src_c])
                    buf1[p, pl.ds(g * L, L)] = vals
                return c2
            lax.fori_loop(0, GCHUNK, pixel_body, jnp.int32(0))

            pltpu.sync_copy(buf1, out_hbm.at[pl.ds(lo, GCHUNK)])
            return carry
        lax.fori_loop(0, nq, chunk_body, jnp.int32(0))


@jax.jit
def _sc_gather(inp2, mask):
    mesh = plsc.VectorSubcoreMesh(core_axis_name="c", subcore_axis_name="s",
                                  num_cores=NC, num_subcores=NS)
    return pl.kernel(
        _sc_body,
        out_type=jax.ShapeDtypeStruct((PIX, C), jnp.float32),
        mesh=mesh,
        compiler_params=pltpu.CompilerParams(needs_layout_passes=False),
        scratch_types=[
            pltpu.VMEM((C,), jnp.float32),
            pltpu.VMEM((C,), jnp.int32),
            pltpu.VMEM((GCHUNK, C), jnp.float32),
            pltpu.VMEM((GCHUNK, C), jnp.float32),
            pltpu.VMEM((GCHUNK, C), jnp.float32),
            pltpu.VMEM((GCHUNK, C), jnp.float32),
            pltpu.SemaphoreType.DMA,
            pltpu.SemaphoreType.DMA,
            pltpu.SemaphoreType.DMA,
            pltpu.SemaphoreType.DMA,
            pltpu.SemaphoreType.DMA,
            pltpu.SemaphoreType.DMA,
            pltpu.SemaphoreType.DMA,
            pltpu.SemaphoreType.DMA,
        ],
    )(inp2, mask)


def kernel(input_tensor, indexes):
    x2 = input_tensor.transpose(0, 2, 3, 1).reshape(PIX, C)
    out2 = _sc_gather(x2, indexes)
    return out2.reshape(N, H, W, C).transpose(0, 3, 1, 2)


# speedup vs baseline: 6.2541x; 1.0725x over previous
import jax
import jax.numpy as jnp
from jax import lax
from jax.experimental import pallas as pl
from jax.experimental.pallas import tpu as pltpu
from jax.experimental.pallas import tpu_sc as plsc

L = 16
N = 4
C = 384
H = 224
W = 224
PIX = N * H * W
NC = 2
NS = 16
NW = NC * NS
PPW = PIX // NW
NBUF = 4
GCHUNK = 64
SCHUNK = 64
SBUF = 2


def _sc_body(inp_hbm, mask_hbm, out_hbm, mask_v, sel_v, shared_v,
             buf0, buf1, buf2, buf3,
             isem0, isem1, isem2, isem3, osem0, osem1, osem2, osem3):
    cid = lax.axis_index("c")
    sid = lax.axis_index("s")
    wid = sid * NC + cid

    pltpu.sync_copy(mask_hbm, mask_v)
    zeros = jnp.zeros((L,), jnp.int32)
    for k in range(C // L):
        sel_v[pl.ds(k * L, L)] = zeros
    count = jnp.int32(0)
    mism = jnp.int32(0)
    for k in range(C // L):
        v = mask_v[pl.ds(k * L, L)]
        nz = v != 0.0
        nzi = nz.astype(jnp.int32)
        cs = plsc.cumsum(nzi)
        pos = count + cs - nzi
        vals = lax.iota(jnp.int32, L) + (k * L)
        plsc.store_scatter(sel_v, [pos], vals, mask=nz)
        count = count + jnp.sum(nzi)
        mism = mism + jnp.sum((~nz).astype(jnp.int32))

    base = wid * PPW
    bufs = (buf0, buf1, buf2, buf3)
    in_sems = (isem0, isem1, isem2, isem3)
    out_sems = (osem0, osem1, osem2, osem3)
    nq = PPW // GCHUNK

    def start_read(q):
        return pltpu.async_copy(
            inp_hbm.at[pl.ds(base + q * GCHUNK, GCHUNK)],
            bufs[q % NBUF], in_sems[q % NBUF])

    def start_write(q):
        return pltpu.async_copy(
            bufs[q % NBUF],
            out_hbm.at[pl.ds(base + q * GCHUNK, GCHUNK)],
            out_sems[q % NBUF])

    @pl.when(mism == 0)
    def _fast():
        snq = PPW // SCHUNK

        def s_read(q):
            return pltpu.async_copy(
                inp_hbm.at[pl.ds(base + q * SCHUNK, SCHUNK)],
                shared_v.at[sid, q % SBUF], in_sems[q % SBUF])

        def s_write(q):
            return pltpu.async_copy(
                shared_v.at[sid, q % SBUF],
                out_hbm.at[pl.ds(base + q * SCHUNK, SCHUNK)],
                out_sems[q % SBUF])

        reads = [None] * snq
        writes = [None] * snq
        for q in range(min(SBUF - 1, snq)):
            reads[q] = s_read(q)
        for q in range(snq):
            if q + SBUF - 1 < snq:
                if q - 1 >= 0:
                    writes[q - 1].wait()
                reads[q + SBUF - 1] = s_read(q + SBUF - 1)
            reads[q].wait()
            writes[q] = s_write(q)
        for r in range(max(0, snq - SBUF), snq):
            writes[r].wait()

    @pl.when(mism != 0)
    def _general():
        def chunk_body(q, carry):
            lo = base + q * GCHUNK
            pltpu.sync_copy(inp_hbm.at[pl.ds(lo, GCHUNK)], buf0)

            def pixel_body(p, c2):
                for g in range(C // L):
                    off = pl.multiple_of(g * L, 8)
                    src_c = sel_v[pl.ds(off, L)]
                    rows = jnp.zeros((L,), jnp.int32) + p
                    vals = plsc.load_gather(buf0, [rows, src_c])
                    buf1[p, pl.ds(g * L, L)] = vals
                return c2
            lax.fori_loop(0, GCHUNK, pixel_body, jnp.int32(0))

            pltpu.sync_copy(buf1, out_hbm.at[pl.ds(lo, GCHUNK)])
            return carry
        lax.fori_loop(0, nq, chunk_body, jnp.int32(0))


@jax.jit
def _sc_gather(inp2, mask):
    mesh = plsc.VectorSubcoreMesh(core_axis_name="c", subcore_axis_name="s",
                                  num_cores=NC, num_subcores=NS)
    return pl.kernel(
        _sc_body,
        out_type=jax.ShapeDtypeStruct((PIX, C), jnp.float32),
        mesh=mesh,
        compiler_params=pltpu.CompilerParams(needs_layout_passes=False),
        scratch_types=[
            pltpu.VMEM((C,), jnp.float32),
            pltpu.VMEM((C,), jnp.int32),
            pltpu.VMEM_SHARED((NS, SBUF, SCHUNK, C), jnp.float32),
            pltpu.VMEM((GCHUNK, C), jnp.float32),
            pltpu.VMEM((GCHUNK, C), jnp.float32),
            pltpu.VMEM((GCHUNK, C), jnp.float32),
            pltpu.VMEM((GCHUNK, C), jnp.float32),
            pltpu.SemaphoreType.DMA,
            pltpu.SemaphoreType.DMA,
            pltpu.SemaphoreType.DMA,
            pltpu.SemaphoreType.DMA,
            pltpu.SemaphoreType.DMA,
            pltpu.SemaphoreType.DMA,
            pltpu.SemaphoreType.DMA,
            pltpu.SemaphoreType.DMA,
        ],
    )(inp2, mask)


def kernel(input_tensor, indexes):
    x2 = input_tensor.transpose(0, 2, 3, 1).reshape(PIX, C)
    out2 = _sc_gather(x2, indexes)
    return out2.reshape(N, H, W, C).transpose(0, 3, 1, 2)


# speedup vs baseline: 6.3032x; 1.0079x over previous
import jax
import jax.numpy as jnp
from jax import lax
from jax.experimental import pallas as pl
from jax.experimental.pallas import tpu as pltpu
from jax.experimental.pallas import tpu_sc as plsc

L = 16
N = 4
C = 384
H = 224
W = 224
PIX = N * H * W
NC = 2
NS = 16
NW = NC * NS
PPW = PIX // NW
NBUF = 4
GCHUNK = 64
SCHUNK = 64
SBUF = 3


def _sc_body(inp_hbm, mask_hbm, out_hbm, mask_v, sel_v, shared_v,
             buf0, buf1,
             isem0, isem1, isem2, isem3, osem0, osem1, osem2, osem3):
    cid = lax.axis_index("c")
    sid = lax.axis_index("s")
    wid = sid * NC + cid

    pltpu.sync_copy(mask_hbm, mask_v)
    zeros = jnp.zeros((L,), jnp.int32)
    for k in range(C // L):
        sel_v[pl.ds(k * L, L)] = zeros
    count = jnp.int32(0)
    mism = jnp.int32(0)
    for k in range(C // L):
        v = mask_v[pl.ds(k * L, L)]
        nz = v != 0.0
        nzi = nz.astype(jnp.int32)
        cs = plsc.cumsum(nzi)
        pos = count + cs - nzi
        vals = lax.iota(jnp.int32, L) + (k * L)
        plsc.store_scatter(sel_v, [pos], vals, mask=nz)
        count = count + jnp.sum(nzi)
        mism = mism + jnp.sum((~nz).astype(jnp.int32))

    base = wid * PPW
    in_sems = (isem0, isem1, isem2, isem3)
    out_sems = (osem0, osem1, osem2, osem3)
    nq = PPW // GCHUNK

    @pl.when(mism == 0)
    def _fast():
        snq = PPW // SCHUNK

        def s_read(q):
            return pltpu.async_copy(
                inp_hbm.at[pl.ds(base + q * SCHUNK, SCHUNK)],
                shared_v.at[sid, q % SBUF], in_sems[q % SBUF])

        def s_write(q):
            return pltpu.async_copy(
                shared_v.at[sid, q % SBUF],
                out_hbm.at[pl.ds(base + q * SCHUNK, SCHUNK)],
                out_sems[q % SBUF])

        reads = [None] * snq
        writes = [None] * snq
        for q in range(min(SBUF - 1, snq)):
            reads[q] = s_read(q)
        for q in range(snq):
            if q + SBUF - 1 < snq:
                if q - 1 >= 0:
                    writes[q - 1].wait()
                reads[q + SBUF - 1] = s_read(q + SBUF - 1)
            reads[q].wait()
            writes[q] = s_write(q)
        for r in range(max(0, snq - SBUF), snq):
            writes[r].wait()

    @pl.when(mism != 0)
    def _general():
        def chunk_body(q, carry):
            lo = base + q * GCHUNK
            pltpu.sync_copy(inp_hbm.at[pl.ds(lo, GCHUNK)], buf0)

            def pixel_body(p, c2):
                for g in range(C // L):
                    off = pl.multiple_of(g * L, 8)
                    src_c = sel_v[pl.ds(off, L)]
                    rows = jnp.zeros((L,), jnp.int32) + p
                    vals = plsc.load_gather(buf0, [rows, src_c])
                    buf1[p, pl.ds(g * L, L)] = vals
                return c2
            lax.fori_loop(0, GCHUNK, pixel_body, jnp.int32(0))

            pltpu.sync_copy(buf1, out_hbm.at[pl.ds(lo, GCHUNK)])
            return carry
        lax.fori_loop(0, nq, chunk_body, jnp.int32(0))


@jax.jit
def _sc_gather(inp2, mask):
    mesh = plsc.VectorSubcoreMesh(core_axis_name="c", subcore_axis_name="s",
                                  num_cores=NC, num_subcores=NS)
    return pl.kernel(
        _sc_body,
        out_type=jax.ShapeDtypeStruct((PIX, C), jnp.float32),
        mesh=mesh,
        compiler_params=pltpu.CompilerParams(needs_layout_passes=False),
        scratch_types=[
            pltpu.VMEM((C,), jnp.float32),
            pltpu.VMEM((C,), jnp.int32),
            pltpu.VMEM_SHARED((NS, SBUF, SCHUNK, C), jnp.float32),
            pltpu.VMEM((GCHUNK, C), jnp.float32),
            pltpu.VMEM((GCHUNK, C), jnp.float32),
            pltpu.SemaphoreType.DMA,
            pltpu.SemaphoreType.DMA,
            pltpu.SemaphoreType.DMA,
            pltpu.SemaphoreType.DMA,
            pltpu.SemaphoreType.DMA,
            pltpu.SemaphoreType.DMA,
            pltpu.SemaphoreType.DMA,
            pltpu.SemaphoreType.DMA,
        ],
    )(inp2, mask)


def kernel(input_tensor, indexes):
    x2 = input_tensor.transpose(0, 2, 3, 1).reshape(PIX, C)
    out2 = _sc_gather(x2, indexes)
    return out2.reshape(N, H, W, C).transpose(0, 3, 1, 2)


# speedup vs baseline: 6.3365x; 1.0053x over previous
import jax
import jax.numpy as jnp
from jax import lax
from jax.experimental import pallas as pl
from jax.experimental.pallas import tpu as pltpu
from jax.experimental.pallas import tpu_sc as plsc

L = 16
N = 4
C = 384
H = 224
W = 224
PIX = N * H * W
NC = 2
NS = 16
NW = NC * NS
PPW = PIX // NW
NBUF = 4
GCHUNK = 32
SCHUNK = 112
SBUF = 2


def _sc_body(inp_hbm, mask_hbm, out_hbm, mask_v, sel_v, shared_v,
             buf0, buf1,
             isem0, isem1, isem2, isem3, osem0, osem1, osem2, osem3):
    cid = lax.axis_index("c")
    sid = lax.axis_index("s")
    wid = sid * NC + cid

    pltpu.sync_copy(mask_hbm, mask_v)
    zeros = jnp.zeros((L,), jnp.int32)
    for k in range(C // L):
        sel_v[pl.ds(k * L, L)] = zeros
    count = jnp.int32(0)
    mism = jnp.int32(0)
    for k in range(C // L):
        v = mask_v[pl.ds(k * L, L)]
        nz = v != 0.0
        nzi = nz.astype(jnp.int32)
        cs = plsc.cumsum(nzi)
        pos = count + cs - nzi
        vals = lax.iota(jnp.int32, L) + (k * L)
        plsc.store_scatter(sel_v, [pos], vals, mask=nz)
        count = count + jnp.sum(nzi)
        mism = mism + jnp.sum((~nz).astype(jnp.int32))

    base = wid * PPW
    in_sems = (isem0, isem1, isem2, isem3)
    out_sems = (osem0, osem1, osem2, osem3)
    nq = PPW // GCHUNK

    @pl.when(mism == 0)
    def _fast():
        snq = PPW // SCHUNK

        def s_read(q):
            return pltpu.async_copy(
                inp_hbm.at[pl.ds(base + q * SCHUNK, SCHUNK)],
                shared_v.at[sid, q % SBUF], in_sems[q % SBUF])

        def s_write(q):
            return pltpu.async_copy(
                shared_v.at[sid, q % SBUF],
                out_hbm.at[pl.ds(base + q * SCHUNK, SCHUNK)],
                out_sems[q % SBUF])

        reads = [None] * snq
        writes = [None] * snq
        for q in range(min(SBUF - 1, snq)):
            reads[q] = s_read(q)
        for q in range(snq):
            if q + SBUF - 1 < snq:
                if q - 1 >= 0:
                    writes[q - 1].wait()
                reads[q + SBUF - 1] = s_read(q + SBUF - 1)
            reads[q].wait()
            writes[q] = s_write(q)
        for r in range(max(0, snq - SBUF), snq):
            writes[r].wait()

    @pl.when(mism != 0)
    def _general():
        def chunk_body(q, carry):
            lo = base + q * GCHUNK
            pltpu.sync_copy(inp_hbm.at[pl.ds(lo, GCHUNK)], buf0)

            def pixel_body(p, c2):
                for g in range(C // L):
                    off = pl.multiple_of(g * L, 8)
                    src_c = sel_v[pl.ds(off, L)]
                    rows = jnp.zeros((L,), jnp.int32) + p
                    vals = plsc.load_gather(buf0, [rows, src_c])
                    buf1[p, pl.ds(g * L, L)] = vals
                return c2
            lax.fori_loop(0, GCHUNK, pixel_body, jnp.int32(0))

            pltpu.sync_copy(buf1, out_hbm.at[pl.ds(lo, GCHUNK)])
            return carry
        lax.fori_loop(0, nq, chunk_body, jnp.int32(0))


@jax.jit
def _sc_gather(inp2, mask):
    mesh = plsc.VectorSubcoreMesh(core_axis_name="c", subcore_axis_name="s",
                                  num_cores=NC, num_subcores=NS)
    return pl.kernel(
        _sc_body,
        out_type=jax.ShapeDtypeStruct((PIX, C), jnp.float32),
        mesh=mesh,
        compiler_params=pltpu.CompilerParams(needs_layout_passes=False),
        scratch_types=[
            pltpu.VMEM((C,), jnp.float32),
            pltpu.VMEM((C,), jnp.int32),
            pltpu.VMEM_SHARED((NS, SBUF, SCHUNK, C), jnp.float32),
            pltpu.VMEM((GCHUNK, C), jnp.float32),
            pltpu.VMEM((GCHUNK, C), jnp.float32),
            pltpu.SemaphoreType.DMA,
            pltpu.SemaphoreType.DMA,
            pltpu.SemaphoreType.DMA,
            pltpu.SemaphoreType.DMA,
            pltpu.SemaphoreType.DMA,
            pltpu.SemaphoreType.DMA,
            pltpu.SemaphoreType.DMA,
            pltpu.SemaphoreType.DMA,
        ],
    )(inp2, mask)


def kernel(input_tensor, indexes):
    x2 = input_tensor.transpose(0, 2, 3, 1).reshape(PIX, C)
    out2 = _sc_gather(x2, indexes)
    return out2.reshape(N, H, W, C).transpose(0, 3, 1, 2)
